# Initial kernel scaffold; baseline (speedup 1.0000x reference)
import jax
import jax.numpy as jnp
from jax.experimental import pallas as pl


def kernel(x, edge_index, edge_weights, W_t, b_t, W1, b1, W2, b2, W_fc, b_fc):
    raise NotImplementedError("write your pallas kernel here")


# speedup vs baseline: 71.8032x; 71.8032x over previous
import jax
import jax.numpy as jnp
from jax.experimental import pallas as pl

_BM = 2000


def _chain_kernel(xin_ref, wcat_ref, w1_ref, w2_ref, wfc_ref,
                  bt_ref, b1_ref, b2_ref, bfc_ref, out_ref):
    xin = xin_ref[...]
    y = jnp.dot(xin, wcat_ref[...], preferred_element_type=jnp.float32)
    y = jax.nn.relu(y + bt_ref[...])
    z = jnp.dot(y, w1_ref[...], preferred_element_type=jnp.float32)
    z = jax.nn.relu(z + b1_ref[...])
    z = jnp.dot(z, w2_ref[...], preferred_element_type=jnp.float32) + b2_ref[...]
    z = jnp.dot(z, wfc_ref[...], preferred_element_type=jnp.float32) + bfc_ref[...]
    out_ref[...] = z


def kernel(x, edge_index, edge_weights, W_t, b_t, W1, b1, W2, b2, W_fc, b_fc):
    B, T, N, C = x.shape
    H = W1.shape[0]
    C_OUT = W_fc.shape[1]

    xl = x[:, T - 1]
    xm1 = jnp.pad(xl, ((0, 0), (1, 0), (0, 0)))[:, :N]
    xp1 = jnp.pad(xl, ((0, 0), (0, 1), (0, 0)))[:, 1:]
    xin = jnp.concatenate([xm1, xl, xp1], axis=-1).reshape(B * N, 3 * C)

    Wcat = jnp.concatenate(
        [W_t[:, :, 0, 0].T, W_t[:, :, 1, 0].T, W_t[:, :, 2, 0].T], axis=0)

    rows = B * N
    grid = (rows // _BM,)
    out = pl.pallas_call(
        _chain_kernel,
        grid=grid,
        in_specs=[
            pl.BlockSpec((_BM, 3 * C), lambda j: (j, 0)),
            pl.BlockSpec((3 * C, H), lambda j: (0, 0)),
            pl.BlockSpec((H, H), lambda j: (0, 0)),
            pl.BlockSpec((H, H), lambda j: (0, 0)),
            pl.BlockSpec((H, C_OUT), lambda j: (0, 0)),
            pl.BlockSpec((1, H), lambda j: (0, 0)),
            pl.BlockSpec((1, H), lambda j: (0, 0)),
            pl.BlockSpec((1, H), lambda j: (0, 0)),
            pl.BlockSpec((1, C_OUT), lambda j: (0, 0)),
        ],
        out_specs=pl.BlockSpec((_BM, C_OUT), lambda j: (j, 0)),
        out_shape=jax.ShapeDtypeStruct((rows, C_OUT), jnp.float32),
    )(xin, Wcat, W1, W2, W_fc,
      b_t.reshape(1, H), b1.reshape(1, H), b2.reshape(1, H),
      b_fc.reshape(1, C_OUT))
    return out.reshape(B, N, C_OUT)
